# baseline (speedup 1.0000x reference)
import functools

import jax
import jax.numpy as jnp
from jax import lax
from jax.experimental import pallas as pl
from jax.experimental.pallas import tpu as pltpu
from jax.experimental.pallas import tpu_sc as plsc

VOCAB = 1000000
EMB = 32
B = 4096
L = 200
NC, NS = 2, 16
NW = NC * NS
B_PER_W = B // NW
NBUF = 8

_mesh = plsc.VectorSubcoreMesh(core_axis_name="c", subcore_axis_name="s")


@functools.partial(
    pl.kernel,
    out_type=jax.ShapeDtypeStruct((B, L, EMB), jnp.float32),
    mesh=_mesh,
    scratch_types=(
        [
            pltpu.VMEM((B_PER_W, L), jnp.int32),
            pltpu.VMEM((NBUF, L, EMB), jnp.float32),
        ]
        + [pltpu.SemaphoreType.DMA] * (2 * NBUF)
    ),
    compiler_params=pltpu.CompilerParams(
        use_tc_tiling_on_sc=False, needs_layout_passes=False
    ),
)
def _gather_kernel(idx_hbm, table_hbm, out_hbm, idx_all, rows, *sems):
    gsem = sems[:NBUF]
    wsem = sems[NBUF:]
    wid = lax.axis_index("s") * NC + lax.axis_index("c")
    base = wid * B_PER_W

    pltpu.sync_copy(idx_hbm.at[pl.ds(base, B_PER_W)], idx_all)

    def gather(c, b):
        return pltpu.make_async_copy(
            table_hbm.at[idx_all.at[c]],
            rows.at[b],
            gsem[b],
        )

    def writeback(c, b):
        return pltpu.make_async_copy(
            rows.at[b],
            out_hbm.at[base + c],
            wsem[b],
        )

    for b in range(NBUF):
        gather(b, b).start()

    @pl.loop(0, B_PER_W - NBUF, step=NBUF)
    def _outer(i):
        for b in range(NBUF):
            c = i + b
            gather(c, b).wait()
            writeback(c, b).start()
            writeback(c, b).wait()
            gather(c + NBUF, b).start()

    for b in range(NBUF):
        c = B_PER_W - NBUF + b
        gather(c, b).wait()
        writeback(c, b).start()
    for b in range(NBUF):
        writeback(B_PER_W - NBUF + b, b).wait()


def kernel(inputs, table):
    return _gather_kernel(inputs.astype(jnp.int32), table)
